# baseline (speedup 1.0000x reference)
import jax
import jax.numpy as jnp
from jax.experimental import pallas as pl
from jax.experimental.pallas import tpu as pltpu

_BATCH = 4
_N_CTX = 4096
_D_MODEL = 2048
_N_EXPERTS = 64

_TILE = 2048


def _router_kernel(x_ref, w_ref, b_ref, out_w_ref, out_e_ref):
    xb = x_ref[...]
    logits = jax.lax.dot_general(
        w_ref[...], xb,
        dimension_numbers=(((1,), (1,)), ((), ())),
        preferred_element_type=jnp.float32,
    )
    logits = logits + b_ref[...]
    m = jnp.max(logits, axis=0, keepdims=True)
    eidx = jax.lax.broadcasted_iota(jnp.int32, logits.shape, 0)
    idx = jnp.min(jnp.where(logits == m, eidx, _N_EXPERTS), axis=0)
    s = jnp.sum(jnp.exp(logits - m), axis=0)
    out_w_ref[0, 0, :] = 1.0 / s
    out_e_ref[0, 0, :] = idx


@jax.jit
def kernel(x, W, b):
    tokens = _BATCH * _N_CTX
    n_tiles = tokens // _TILE
    xf = x.reshape(tokens, _D_MODEL)
    b2 = jnp.broadcast_to(b[:, None], (_N_EXPERTS, _TILE))

    grid = (n_tiles,)
    weights, experts = pl.pallas_call(
        _router_kernel,
        grid=grid,
        in_specs=[
            pl.BlockSpec((_TILE, _D_MODEL), lambda i: (i, 0)),
            pl.BlockSpec((_N_EXPERTS, _D_MODEL), lambda i: (0, 0)),
            pl.BlockSpec((_N_EXPERTS, _TILE), lambda i: (0, 0)),
        ],
        out_specs=[
            pl.BlockSpec((1, 1, _TILE), lambda i: (i, 0, 0)),
            pl.BlockSpec((1, 1, _TILE), lambda i: (i, 0, 0)),
        ],
        out_shape=[
            jax.ShapeDtypeStruct((n_tiles, 1, _TILE), jnp.float32),
            jax.ShapeDtypeStruct((n_tiles, 1, _TILE), jnp.int32),
        ],
        compiler_params=pltpu.CompilerParams(
            dimension_semantics=("arbitrary",),
        ),
    )(xf, W, b2)

    weights = weights.reshape(_BATCH, _N_CTX)
    experts = experts.reshape(_BATCH, _N_CTX)
    return (weights, experts)


# speedup vs baseline: 1.0087x; 1.0087x over previous
import jax
import jax.numpy as jnp
from jax.experimental import pallas as pl
from jax.experimental.pallas import tpu as pltpu

_BATCH = 4
_N_CTX = 4096
_D_MODEL = 2048
_N_EXPERTS = 64

_TILE = 1024


def _router_kernel(x_ref, w_ref, b_ref, out_w_ref, out_e_ref):
    xb = x_ref[...]
    logits = jax.lax.dot_general(
        w_ref[...], xb,
        dimension_numbers=(((1,), (1,)), ((), ())),
        preferred_element_type=jnp.float32,
    )
    logits = logits + b_ref[...]
    m = jnp.max(logits, axis=0, keepdims=True)
    eidx = jax.lax.broadcasted_iota(jnp.int32, logits.shape, 0)
    idx = jnp.min(jnp.where(logits == m, eidx, _N_EXPERTS), axis=0)
    s = jnp.sum(jnp.exp(logits - m), axis=0)
    out_w_ref[0, 0, :] = 1.0 / s
    out_e_ref[0, 0, :] = idx


@jax.jit
def kernel(x, W, b):
    tokens = _BATCH * _N_CTX
    n_tiles = tokens // _TILE
    xf = x.reshape(tokens, _D_MODEL)
    b2 = jnp.broadcast_to(b[:, None], (_N_EXPERTS, _TILE))

    grid = (n_tiles,)
    weights, experts = pl.pallas_call(
        _router_kernel,
        grid=grid,
        in_specs=[
            pl.BlockSpec((_TILE, _D_MODEL), lambda i: (i, 0)),
            pl.BlockSpec((_N_EXPERTS, _D_MODEL), lambda i: (0, 0)),
            pl.BlockSpec((_N_EXPERTS, _TILE), lambda i: (0, 0)),
        ],
        out_specs=[
            pl.BlockSpec((1, 1, _TILE), lambda i: (i, 0, 0)),
            pl.BlockSpec((1, 1, _TILE), lambda i: (i, 0, 0)),
        ],
        out_shape=[
            jax.ShapeDtypeStruct((n_tiles, 1, _TILE), jnp.float32),
            jax.ShapeDtypeStruct((n_tiles, 1, _TILE), jnp.int32),
        ],
        compiler_params=pltpu.CompilerParams(
            dimension_semantics=("arbitrary",),
        ),
    )(xf, W, b2)

    weights = weights.reshape(_BATCH, _N_CTX)
    experts = experts.reshape(_BATCH, _N_CTX)
    return (weights, experts)


# speedup vs baseline: 1.0241x; 1.0153x over previous
import jax
import jax.numpy as jnp
from jax.experimental import pallas as pl
from jax.experimental.pallas import tpu as pltpu

_BATCH = 4
_N_CTX = 4096
_D_MODEL = 2048
_N_EXPERTS = 64

_TILE = 1024


def _router_kernel(x_ref, w_ref, b_ref, out_w_ref, out_e_ref):
    xb = x_ref[...]
    logits = jax.lax.dot_general(
        w_ref[...], xb,
        dimension_numbers=(((1,), (1,)), ((), ())),
        preferred_element_type=jnp.float32,
    )
    logits = logits + b_ref[...]
    m = jnp.max(logits, axis=0, keepdims=True)
    eidx = jax.lax.broadcasted_iota(jnp.int32, logits.shape, 0)
    idx = jnp.min(jnp.where(logits == m, eidx, _N_EXPERTS), axis=0)
    s = jnp.sum(jnp.exp(logits - m), axis=0)
    out_w_ref[0, 0, :] = 1.0 / s
    out_e_ref[0, 0, :] = idx


@jax.jit
def kernel(x, W, b):
    tokens = _BATCH * _N_CTX
    n_tiles = tokens // _TILE
    xf = x.reshape(tokens, _D_MODEL)
    b2 = b.reshape(_N_EXPERTS, 1)

    grid = (n_tiles,)
    weights, experts = pl.pallas_call(
        _router_kernel,
        grid=grid,
        in_specs=[
            pl.BlockSpec((_TILE, _D_MODEL), lambda i: (i, 0)),
            pl.BlockSpec((_N_EXPERTS, _D_MODEL), lambda i: (0, 0)),
            pl.BlockSpec((_N_EXPERTS, 1), lambda i: (0, 0)),
        ],
        out_specs=[
            pl.BlockSpec((1, 1, _TILE), lambda i: (i, 0, 0)),
            pl.BlockSpec((1, 1, _TILE), lambda i: (i, 0, 0)),
        ],
        out_shape=[
            jax.ShapeDtypeStruct((n_tiles, 1, _TILE), jnp.float32),
            jax.ShapeDtypeStruct((n_tiles, 1, _TILE), jnp.int32),
        ],
        compiler_params=pltpu.CompilerParams(
            dimension_semantics=("arbitrary",),
        ),
    )(xf, W, b2)

    weights = weights.reshape(_BATCH, _N_CTX)
    experts = experts.reshape(_BATCH, _N_CTX)
    return (weights, experts)
